# baseline (speedup 1.0000x reference)
import functools

import jax
import jax.numpy as jnp
from jax.experimental import pallas as pl
from jax.experimental.pallas import tpu as pltpu
from jax.experimental.pallas import tpu_sc as plsc

_E = 8
_K = 2
_T = 4096
_D = 768
_H = _D // 2
_CAP = int(_T * _K / _E * 1.25)
_NSLOT = _E * _CAP
_NPAIR = _T * _K

_BT_R = 512
_BC = 256
_NBLK = _NSLOT // _BC

_NW = 32
_TPW = _T // _NW
_CHD = 64
_CHT = 32


def _router_body(x_ref, wg_ref, lt_ref, d0_ref, d1_ref, s0_ref, s1_ref, cpa_ref, cpb_ref, carry_ref):
    b = pl.program_id(0)

    @pl.when(b == 0)
    def _():
        carry_ref[...] = jnp.zeros_like(carry_ref)

    x = x_ref[...]
    logits = jnp.dot(x, wg_ref[...], preferred_element_type=jnp.float32)

    iota_e = jax.lax.broadcasted_iota(jnp.int32, logits.shape, 1)
    m1 = jnp.max(logits, axis=1, keepdims=True)
    i1 = jnp.min(jnp.where(logits == m1, iota_e, _E), axis=1, keepdims=True)
    sel1 = iota_e == i1
    l2 = jnp.where(sel1, -jnp.inf, logits)
    m2 = jnp.max(l2, axis=1, keepdims=True)
    i2 = jnp.min(jnp.where(l2 == m2, iota_e, _E), axis=1, keepdims=True)
    sel2 = iota_e == i2

    g1 = 1.0 / (1.0 + jnp.exp(m2 - m1))
    g2 = 1.0 - g1

    cnt = sel1.astype(jnp.float32) + sel2.astype(jnp.float32)
    cum = jnp.dot(lt_ref[...], cnt,
                  preferred_element_type=jnp.float32) + carry_ref[...]
    carry_ref[...] += jnp.sum(cnt, axis=0, keepdims=True)

    pos1 = jnp.sum(jnp.where(sel1, cum, 0.0), axis=1, keepdims=True).astype(jnp.int32)
    pos2 = jnp.sum(jnp.where(sel2, cum, 0.0), axis=1, keepdims=True).astype(jnp.int32)
    keep1 = pos1 < _CAP
    keep2 = pos2 < _CAP

    slot1 = i1 * _CAP + pos1
    slot2 = i2 * _CAP + pos2
    dst1 = jnp.where(keep1, slot1, _NSLOT)
    dst2 = jnp.where(keep2, slot2, _NSLOT)
    src1 = jnp.where(keep1, slot1, _NSLOT)
    src2 = jnp.where(keep2, slot2, _NSLOT)
    c1 = jnp.where(keep1, g1, 0.0)
    c2 = jnp.where(keep2, g2, 0.0)

    d0_ref[...] = dst1
    d1_ref[...] = dst2
    s0_ref[...] = src1
    s1_ref[...] = src2
    cpa_ref[...] = jnp.broadcast_to(c1, (_BT_R, 128))
    cpb_ref[...] = jnp.broadcast_to(c2, (_BT_R, 128))


def _expert_body(buf_ref, w1_ref, w2_ref, cs_ref, y_ref):
    i = pl.program_id(0)

    @pl.when(i < _NBLK)
    def _():
        xb = buf_ref[...].astype(jnp.bfloat16)
        w1 = w1_ref[0].astype(jnp.bfloat16)
        w2 = w2_ref[0].astype(jnp.bfloat16)
        h = jnp.maximum(jnp.dot(xb, w1, preferred_element_type=jnp.float32), 0.0)
        h2 = (h * cs_ref[:, 0:1]).astype(jnp.bfloat16)
        y_ref[...] = jnp.maximum(
            jnp.dot(h2, w2, preferred_element_type=jnp.float32), 0.0)

    @pl.when(i == _NBLK)
    def _():
        y_ref[...] = jnp.zeros_like(y_ref)


def _dispatch_sc(x_hbm, d0_hbm, d1_hbm, cpa_hbm, cpb_hbm, buf_hbm, cslot_hbm,
                 d0v, d1v, rows, cv0, cv1, sem1, sem2):
    wid = jax.lax.axis_index("s") * 2 + jax.lax.axis_index("c")
    for ci in range(_TPW // _CHD):
        t0 = wid * _TPW + ci * _CHD
        pltpu.sync_copy(d0_hbm.at[pl.ds(t0, _CHD)], d0v)
        pltpu.sync_copy(d1_hbm.at[pl.ds(t0, _CHD)], d1v)
        cpx = pltpu.async_copy(x_hbm.at[pl.ds(t0, _CHD)], rows, sem1)
        cpa = pltpu.async_copy(cpa_hbm.at[pl.ds(t0, _CHD)], cv0, sem1)
        cpb = pltpu.async_copy(cpb_hbm.at[pl.ds(t0, _CHD)], cv1, sem1)
        cpx.wait(); cpa.wait(); cpb.wait()
        s0 = pltpu.async_copy(rows, buf_hbm.at[d0v], sem2)
        s1 = pltpu.async_copy(rows, buf_hbm.at[d1v], sem2)
        s2 = pltpu.async_copy(cv0, cslot_hbm.at[d0v], sem2)
        s3 = pltpu.async_copy(cv1, cslot_hbm.at[d1v], sem2)
        s0.wait(); s1.wait(); s2.wait(); s3.wait()


def _combine_sc(x_hbm, y_hbm, s0_hbm, s1_hbm, out_hbm,
                s0v, s1v, acc, yv0, yv1, sem1, sem2):
    wid = jax.lax.axis_index("s") * 2 + jax.lax.axis_index("c")

    def _load_src(ci, buf_a, buf_b):
        t0 = wid * _TPW + ci * _CHT
        pltpu.sync_copy(s0_hbm.at[pl.ds(t0, _CHT)], s0v)
        pltpu.sync_copy(s1_hbm.at[pl.ds(t0, _CHT)], s1v)
        ga = pltpu.async_copy(y_hbm.at[s0v], buf_a, sem1)
        gb = pltpu.async_copy(y_hbm.at[s1v], buf_b, sem2)
        return ga, gb


    nch = _TPW // _CHT
    ga, gb = _load_src(0, yv0, yv1)
    for ci in range(nch):
        t0 = wid * _TPW + ci * _CHT
        pltpu.sync_copy(x_hbm.at[pl.ds(t0, _CHT)], acc)
        ga.wait()
        gb.wait()
        @plsc.parallel_loop(0, _CHT, 1, unroll=4)
        def _acc_rows(r):
            for c in range(_D // 16):
                sl = (r, pl.ds(c * 16, 16))
                acc[sl] = acc[sl] + yv0[sl] + yv1[sl]
        pltpu.sync_copy(acc, out_hbm.at[pl.ds(t0, _CHT)])
        if ci + 1 < nch:
            ga, gb = _load_src(ci + 1, yv0, yv1)


def kernel(x, Wg, W1, W2):
    ii = jax.lax.broadcasted_iota(jnp.int32, (_BT_R, _BT_R), 0)
    jj = jax.lax.broadcasted_iota(jnp.int32, (_BT_R, _BT_R), 1)
    lt = (jj < ii).astype(jnp.float32)
    d0, d1, s0, s1, cpa, cpb = pl.pallas_call(
        _router_body,
        grid=(_T // _BT_R,),
        in_specs=[
            pl.BlockSpec((_BT_R, _D), lambda b: (b, 0)),
            pl.BlockSpec((_D, _E), lambda b: (0, 0)),
            pl.BlockSpec((_BT_R, _BT_R), lambda b: (0, 0)),
        ],
        out_specs=[
            pl.BlockSpec((_BT_R, 1), lambda b: (b, 0)),
            pl.BlockSpec((_BT_R, 1), lambda b: (b, 0)),
            pl.BlockSpec((_BT_R, 1), lambda b: (b, 0)),
            pl.BlockSpec((_BT_R, 1), lambda b: (b, 0)),
            pl.BlockSpec((_BT_R, 128), lambda b: (b, 0)),
            pl.BlockSpec((_BT_R, 128), lambda b: (b, 0)),
        ],
        out_shape=[
            jax.ShapeDtypeStruct((_T, 1), jnp.int32),
            jax.ShapeDtypeStruct((_T, 1), jnp.int32),
            jax.ShapeDtypeStruct((_T, 1), jnp.int32),
            jax.ShapeDtypeStruct((_T, 1), jnp.int32),
            jax.ShapeDtypeStruct((_T, 128), jnp.float32),
            jax.ShapeDtypeStruct((_T, 128), jnp.float32),
        ],
        scratch_shapes=[pltpu.VMEM((1, _E), jnp.float32)],
    )(x, Wg, lt)

    mesh = plsc.VectorSubcoreMesh(core_axis_name="c", subcore_axis_name="s")

    dispatch = functools.partial(
        pl.kernel,
        mesh=mesh,
        out_type=[
            jax.ShapeDtypeStruct((_NSLOT + _BC, _D), jnp.float32),
            jax.ShapeDtypeStruct((_NSLOT + _BC, 128), jnp.float32),
        ],
        scratch_types=[
            pltpu.VMEM((_CHD,), jnp.int32),
            pltpu.VMEM((_CHD,), jnp.int32),
            pltpu.VMEM((_CHD, _D), jnp.float32),
            pltpu.VMEM((_CHD, 128), jnp.float32),
            pltpu.VMEM((_CHD, 128), jnp.float32),
            pltpu.SemaphoreType.DMA,
            pltpu.SemaphoreType.DMA,
        ],
    )(_dispatch_sc)
    d0f = d0.reshape(_T)
    d1f = d1.reshape(_T)
    s0f = s0.reshape(_T)
    s1f = s1.reshape(_T)
    buf, cslot = dispatch(x, d0f, d1f, cpa, cpb)

    y = pl.pallas_call(
        _expert_body,
        grid=(_NBLK + 1,),
        in_specs=[
            pl.BlockSpec((_BC, _D), lambda i: (i, 0)),
            pl.BlockSpec((1, _D, _H), lambda i: (jnp.minimum(i // (_CAP // _BC), _E - 1), 0, 0)),
            pl.BlockSpec((1, _H, _D), lambda i: (jnp.minimum(i // (_CAP // _BC), _E - 1), 0, 0)),
            pl.BlockSpec((_BC, 128), lambda i: (i, 0)),
        ],
        out_specs=pl.BlockSpec((_BC, _D), lambda i: (i, 0)),
        out_shape=jax.ShapeDtypeStruct((_NSLOT + _BC, _D), jnp.float32),
    )(buf, W1, W2, cslot)

    combine = functools.partial(
        pl.kernel,
        mesh=mesh,
        out_type=jax.ShapeDtypeStruct((_T, _D), jnp.float32),
        scratch_types=[
            pltpu.VMEM((_CHT,), jnp.int32),
            pltpu.VMEM((_CHT,), jnp.int32),
            pltpu.VMEM((_CHT, _D), jnp.float32),
            pltpu.VMEM((_CHT, _D), jnp.float32),
            pltpu.VMEM((_CHT, _D), jnp.float32),
            pltpu.SemaphoreType.DMA,
            pltpu.SemaphoreType.DMA,
        ],
    )(_combine_sc)
    out = combine(x, y, s0f, s1f)
    return out


# speedup vs baseline: 1.0079x; 1.0079x over previous
import functools

import jax
import jax.numpy as jnp
from jax.experimental import pallas as pl
from jax.experimental.pallas import tpu as pltpu
from jax.experimental.pallas import tpu_sc as plsc

_E = 8
_K = 2
_T = 4096
_D = 768
_H = _D // 2
_CAP = int(_T * _K / _E * 1.25)
_NSLOT = _E * _CAP
_NPAIR = _T * _K

_BT_R = 512
_BC = 256
_NBLK = _NSLOT // _BC

_NW = 32
_TPW = _T // _NW
_CHD = 64
_CHT = 32


def _router_body(x_ref, wg_ref, d0_ref, d1_ref, s0_ref, s1_ref, cpa_ref, cpb_ref, carry_ref):
    b = pl.program_id(0)

    @pl.when(b == 0)
    def _():
        carry_ref[...] = jnp.zeros_like(carry_ref)

    x = x_ref[...]
    logits = jnp.dot(x, wg_ref[...], preferred_element_type=jnp.float32)

    iota_e = jax.lax.broadcasted_iota(jnp.int32, logits.shape, 1)
    m1 = jnp.max(logits, axis=1, keepdims=True)
    i1 = jnp.min(jnp.where(logits == m1, iota_e, _E), axis=1, keepdims=True)
    sel1 = iota_e == i1
    l2 = jnp.where(sel1, -jnp.inf, logits)
    m2 = jnp.max(l2, axis=1, keepdims=True)
    i2 = jnp.min(jnp.where(l2 == m2, iota_e, _E), axis=1, keepdims=True)
    sel2 = iota_e == i2

    g1 = 1.0 / (1.0 + jnp.exp(m2 - m1))
    g2 = 1.0 - g1

    cnt = sel1.astype(jnp.float32) + sel2.astype(jnp.float32)
    ii = jax.lax.broadcasted_iota(jnp.int32, (_BT_R, _BT_R), 0)
    jj = jax.lax.broadcasted_iota(jnp.int32, (_BT_R, _BT_R), 1)
    lt = (jj < ii).astype(jnp.float32)
    cum = jnp.dot(lt, cnt, preferred_element_type=jnp.float32) + carry_ref[...]
    carry_ref[...] += jnp.sum(cnt, axis=0, keepdims=True)

    pos1 = jnp.sum(jnp.where(sel1, cum, 0.0), axis=1, keepdims=True).astype(jnp.int32)
    pos2 = jnp.sum(jnp.where(sel2, cum, 0.0), axis=1, keepdims=True).astype(jnp.int32)
    keep1 = pos1 < _CAP
    keep2 = pos2 < _CAP

    slot1 = i1 * _CAP + pos1
    slot2 = i2 * _CAP + pos2
    dst1 = jnp.where(keep1, slot1, _NSLOT)
    dst2 = jnp.where(keep2, slot2, _NSLOT)
    src1 = jnp.where(keep1, slot1, _NSLOT)
    src2 = jnp.where(keep2, slot2, _NSLOT)
    c1 = jnp.where(keep1, g1, 0.0)
    c2 = jnp.where(keep2, g2, 0.0)

    d0_ref[...] = dst1
    d1_ref[...] = dst2
    s0_ref[...] = src1
    s1_ref[...] = src2
    cpa_ref[...] = jnp.broadcast_to(c1, (_BT_R, 128))
    cpb_ref[...] = jnp.broadcast_to(c2, (_BT_R, 128))


def _expert_body(buf_ref, w1_ref, w2_ref, cs_ref, y_ref):
    i = pl.program_id(0)

    @pl.when(i < _NBLK)
    def _():
        xb = buf_ref[...].astype(jnp.bfloat16)
        w1 = w1_ref[0].astype(jnp.bfloat16)
        w2 = w2_ref[0].astype(jnp.bfloat16)
        h = jnp.maximum(jnp.dot(xb, w1, preferred_element_type=jnp.float32), 0.0)
        h2 = (h * cs_ref[:, 0:1]).astype(jnp.bfloat16)
        y_ref[...] = jnp.maximum(
            jnp.dot(h2, w2, preferred_element_type=jnp.float32), 0.0)

    @pl.when(i == _NBLK)
    def _():
        y_ref[...] = jnp.zeros_like(y_ref)


def _dispatch_sc(x_hbm, d0_hbm, d1_hbm, cpa_hbm, cpb_hbm, buf_hbm, cslot_hbm,
                 d0v, d1v, rows, cv0, cv1, sem1, sem2):
    wid = jax.lax.axis_index("s") * 2 + jax.lax.axis_index("c")
    for ci in range(_TPW // _CHD):
        t0 = wid * _TPW + ci * _CHD
        pltpu.sync_copy(d0_hbm.at[pl.ds(t0, _CHD)], d0v)
        pltpu.sync_copy(d1_hbm.at[pl.ds(t0, _CHD)], d1v)
        cpx = pltpu.async_copy(x_hbm.at[pl.ds(t0, _CHD)], rows, sem1)
        cpa = pltpu.async_copy(cpa_hbm.at[pl.ds(t0, _CHD)], cv0, sem1)
        cpb = pltpu.async_copy(cpb_hbm.at[pl.ds(t0, _CHD)], cv1, sem1)
        cpx.wait(); cpa.wait(); cpb.wait()
        s0 = pltpu.async_copy(rows, buf_hbm.at[d0v], sem2)
        s1 = pltpu.async_copy(rows, buf_hbm.at[d1v], sem2)
        s2 = pltpu.async_copy(cv0, cslot_hbm.at[d0v], sem2)
        s3 = pltpu.async_copy(cv1, cslot_hbm.at[d1v], sem2)
        s0.wait(); s1.wait(); s2.wait(); s3.wait()


def _combine_sc(x_hbm, y_hbm, s0_hbm, s1_hbm, out_hbm,
                s0v, s1v, acc, yv0, yv1, sem1, sem2):
    wid = jax.lax.axis_index("s") * 2 + jax.lax.axis_index("c")

    def _load_src(ci, buf_a, buf_b):
        t0 = wid * _TPW + ci * _CHT
        pltpu.sync_copy(s0_hbm.at[pl.ds(t0, _CHT)], s0v)
        pltpu.sync_copy(s1_hbm.at[pl.ds(t0, _CHT)], s1v)
        ga = pltpu.async_copy(y_hbm.at[s0v], buf_a, sem1)
        gb = pltpu.async_copy(y_hbm.at[s1v], buf_b, sem2)
        return ga, gb


    nch = _TPW // _CHT
    ga, gb = _load_src(0, yv0, yv1)
    for ci in range(nch):
        t0 = wid * _TPW + ci * _CHT
        pltpu.sync_copy(x_hbm.at[pl.ds(t0, _CHT)], acc)
        ga.wait()
        gb.wait()
        @plsc.parallel_loop(0, _CHT, 1, unroll=4)
        def _acc_rows(r):
            for c in range(_D // 16):
                sl = (r, pl.ds(c * 16, 16))
                acc[sl] = acc[sl] + yv0[sl] + yv1[sl]
        pltpu.sync_copy(acc, out_hbm.at[pl.ds(t0, _CHT)])
        if ci + 1 < nch:
            ga, gb = _load_src(ci + 1, yv0, yv1)


def kernel(x, Wg, W1, W2):
    d0, d1, s0, s1, cpa, cpb = pl.pallas_call(
        _router_body,
        grid=(_T // _BT_R,),
        in_specs=[
            pl.BlockSpec((_BT_R, _D), lambda b: (b, 0)),
            pl.BlockSpec((_D, _E), lambda b: (0, 0)),
        ],
        out_specs=[
            pl.BlockSpec((_BT_R, 1), lambda b: (b, 0)),
            pl.BlockSpec((_BT_R, 1), lambda b: (b, 0)),
            pl.BlockSpec((_BT_R, 1), lambda b: (b, 0)),
            pl.BlockSpec((_BT_R, 1), lambda b: (b, 0)),
            pl.BlockSpec((_BT_R, 128), lambda b: (b, 0)),
            pl.BlockSpec((_BT_R, 128), lambda b: (b, 0)),
        ],
        out_shape=[
            jax.ShapeDtypeStruct((_T, 1), jnp.int32),
            jax.ShapeDtypeStruct((_T, 1), jnp.int32),
            jax.ShapeDtypeStruct((_T, 1), jnp.int32),
            jax.ShapeDtypeStruct((_T, 1), jnp.int32),
            jax.ShapeDtypeStruct((_T, 128), jnp.float32),
            jax.ShapeDtypeStruct((_T, 128), jnp.float32),
        ],
        scratch_shapes=[pltpu.VMEM((1, _E), jnp.float32)],
    )(x, Wg)

    mesh = plsc.VectorSubcoreMesh(core_axis_name="c", subcore_axis_name="s")

    dispatch = functools.partial(
        pl.kernel,
        mesh=mesh,
        out_type=[
            jax.ShapeDtypeStruct((_NSLOT + _BC, _D), jnp.float32),
            jax.ShapeDtypeStruct((_NSLOT + _BC, 128), jnp.float32),
        ],
        scratch_types=[
            pltpu.VMEM((_CHD,), jnp.int32),
            pltpu.VMEM((_CHD,), jnp.int32),
            pltpu.VMEM((_CHD, _D), jnp.float32),
            pltpu.VMEM((_CHD, 128), jnp.float32),
            pltpu.VMEM((_CHD, 128), jnp.float32),
            pltpu.SemaphoreType.DMA,
            pltpu.SemaphoreType.DMA,
        ],
    )(_dispatch_sc)
    d0f = d0.reshape(_T)
    d1f = d1.reshape(_T)
    s0f = s0.reshape(_T)
    s1f = s1.reshape(_T)
    buf, cslot = dispatch(x, d0f, d1f, cpa, cpb)

    y = pl.pallas_call(
        _expert_body,
        grid=(_NBLK + 1,),
        in_specs=[
            pl.BlockSpec((_BC, _D), lambda i: (i, 0)),
            pl.BlockSpec((1, _D, _H), lambda i: (jnp.minimum(i // (_CAP // _BC), _E - 1), 0, 0)),
            pl.BlockSpec((1, _H, _D), lambda i: (jnp.minimum(i // (_CAP // _BC), _E - 1), 0, 0)),
            pl.BlockSpec((_BC, 128), lambda i: (i, 0)),
        ],
        out_specs=pl.BlockSpec((_BC, _D), lambda i: (i, 0)),
        out_shape=jax.ShapeDtypeStruct((_NSLOT + _BC, _D), jnp.float32),
    )(buf, W1, W2, cslot)

    combine = functools.partial(
        pl.kernel,
        mesh=mesh,
        out_type=jax.ShapeDtypeStruct((_T, _D), jnp.float32),
        scratch_types=[
            pltpu.VMEM((_CHT,), jnp.int32),
            pltpu.VMEM((_CHT,), jnp.int32),
            pltpu.VMEM((_CHT, _D), jnp.float32),
            pltpu.VMEM((_CHT, _D), jnp.float32),
            pltpu.VMEM((_CHT, _D), jnp.float32),
            pltpu.SemaphoreType.DMA,
            pltpu.SemaphoreType.DMA,
        ],
    )(_combine_sc)
    out = combine(x, y, s0f, s1f)
    return out


# speedup vs baseline: 1.0124x; 1.0044x over previous
import functools

import jax
import jax.numpy as jnp
from jax.experimental import pallas as pl
from jax.experimental.pallas import tpu as pltpu
from jax.experimental.pallas import tpu_sc as plsc

_E = 8
_K = 2
_T = 4096
_D = 768
_H = _D // 2
_CAP = int(_T * _K / _E * 1.25)
_NSLOT = _E * _CAP
_NPAIR = _T * _K

_BT_R = 512
_BC = 256
_NBLK = _NSLOT // _BC

_NW = 32
_TPW = _T // _NW
_CHD = 64
_CHT = 32


def _router_body(x_ref, wg_ref, d0_ref, d1_ref, s0_ref, s1_ref, cpa_ref, cpb_ref, carry_ref):
    b = pl.program_id(0)

    @pl.when(b == 0)
    def _():
        carry_ref[...] = jnp.zeros_like(carry_ref)

    x = x_ref[...]
    logits = jnp.dot(x, wg_ref[...], preferred_element_type=jnp.float32)

    iota_e = jax.lax.broadcasted_iota(jnp.int32, logits.shape, 1)
    m1 = jnp.max(logits, axis=1, keepdims=True)
    i1 = jnp.min(jnp.where(logits == m1, iota_e, _E), axis=1, keepdims=True)
    sel1 = iota_e == i1
    l2 = jnp.where(sel1, -jnp.inf, logits)
    m2 = jnp.max(l2, axis=1, keepdims=True)
    i2 = jnp.min(jnp.where(l2 == m2, iota_e, _E), axis=1, keepdims=True)
    sel2 = iota_e == i2

    g1 = 1.0 / (1.0 + jnp.exp(m2 - m1))
    g2 = 1.0 - g1

    cnt = sel1.astype(jnp.float32) + sel2.astype(jnp.float32)
    ii = jax.lax.broadcasted_iota(jnp.int32, (_BT_R, _BT_R), 0)
    jj = jax.lax.broadcasted_iota(jnp.int32, (_BT_R, _BT_R), 1)
    lt = (jj < ii).astype(jnp.float32)
    cum = jnp.dot(lt, cnt, preferred_element_type=jnp.float32) + carry_ref[...]
    carry_ref[...] += jnp.sum(cnt, axis=0, keepdims=True)

    pos1 = jnp.sum(jnp.where(sel1, cum, 0.0), axis=1, keepdims=True).astype(jnp.int32)
    pos2 = jnp.sum(jnp.where(sel2, cum, 0.0), axis=1, keepdims=True).astype(jnp.int32)
    keep1 = pos1 < _CAP
    keep2 = pos2 < _CAP

    slot1 = i1 * _CAP + pos1
    slot2 = i2 * _CAP + pos2
    dst1 = jnp.where(keep1, slot1, _NSLOT)
    dst2 = jnp.where(keep2, slot2, _NSLOT)
    src1 = jnp.where(keep1, slot1, _NSLOT)
    src2 = jnp.where(keep2, slot2, _NSLOT)
    c1 = jnp.where(keep1, g1, 0.0)
    c2 = jnp.where(keep2, g2, 0.0)

    d0_ref[...] = dst1
    d1_ref[...] = dst2
    s0_ref[...] = src1
    s1_ref[...] = src2
    cpa_ref[...] = jnp.broadcast_to(c1, (_BT_R, 128))
    cpb_ref[...] = jnp.broadcast_to(c2, (_BT_R, 128))


def _expert_body(buf_ref, w1_ref, w2_ref, cs_ref, y_ref):
    i = pl.program_id(0)

    @pl.when(i < _NBLK)
    def _():
        xb = buf_ref[...].astype(jnp.bfloat16)
        w1 = w1_ref[0].astype(jnp.bfloat16)
        w2 = w2_ref[0].astype(jnp.bfloat16)
        h = jnp.maximum(jnp.dot(xb, w1, preferred_element_type=jnp.float32), 0.0)
        h2 = (h * cs_ref[:, 0:1]).astype(jnp.bfloat16)
        y_ref[...] = jnp.maximum(
            jnp.dot(h2, w2, preferred_element_type=jnp.float32), 0.0)

    @pl.when(i == _NBLK)
    def _():
        y_ref[...] = jnp.zeros_like(y_ref)


def _dispatch_sc(x_hbm, d0_hbm, d1_hbm, cpa_hbm, cpb_hbm, buf_hbm, cslot_hbm,
                 d0v, d1v, rows, cv0, cv1, sem1, sem2):
    wid = jax.lax.axis_index("s") * 2 + jax.lax.axis_index("c")
    for ci in range(_TPW // _CHD):
        t0 = wid * _TPW + ci * _CHD
        pltpu.sync_copy(d0_hbm.at[pl.ds(t0, _CHD)], d0v)
        pltpu.sync_copy(d1_hbm.at[pl.ds(t0, _CHD)], d1v)
        cpx = pltpu.async_copy(x_hbm.at[pl.ds(t0, _CHD)], rows, sem1)
        cpa = pltpu.async_copy(cpa_hbm.at[pl.ds(t0, _CHD)], cv0, sem1)
        cpb = pltpu.async_copy(cpb_hbm.at[pl.ds(t0, _CHD)], cv1, sem1)
        cpx.wait(); cpa.wait(); cpb.wait()
        s0 = pltpu.async_copy(rows, buf_hbm.at[d0v], sem2)
        s1 = pltpu.async_copy(rows, buf_hbm.at[d1v], sem2)
        s2 = pltpu.async_copy(cv0, cslot_hbm.at[d0v], sem2)
        s3 = pltpu.async_copy(cv1, cslot_hbm.at[d1v], sem2)
        s0.wait(); s1.wait(); s2.wait(); s3.wait()


def _combine_sc(x_hbm, y_hbm, s0_hbm, s1_hbm, out_hbm,
                s0a, s1a, s0b, s1b, acc, y0a, y1a, y0b, y1b,
                semA0, semA1, semB0, semB1):
    wid = jax.lax.axis_index("s") * 2 + jax.lax.axis_index("c")
    nch = _TPW // _CHT

    def _load_src(ci):
        if ci % 2 == 0:
            s0v, s1v, b0, b1, m0, m1 = s0a, s1a, y0a, y1a, semA0, semA1
        else:
            s0v, s1v, b0, b1, m0, m1 = s0b, s1b, y0b, y1b, semB0, semB1
        t0 = wid * _TPW + ci * _CHT
        pltpu.sync_copy(s0_hbm.at[pl.ds(t0, _CHT)], s0v)
        pltpu.sync_copy(s1_hbm.at[pl.ds(t0, _CHT)], s1v)
        ga = pltpu.async_copy(y_hbm.at[s0v], b0, m0)
        gb = pltpu.async_copy(y_hbm.at[s1v], b1, m1)
        return ga, gb

    pending = _load_src(0)
    for ci in range(nch):
        nxt = _load_src(ci + 1) if ci + 1 < nch else None
        t0 = wid * _TPW + ci * _CHT
        pltpu.sync_copy(x_hbm.at[pl.ds(t0, _CHT)], acc)
        ga, gb = pending
        ga.wait()
        gb.wait()
        yv0 = y0a if ci % 2 == 0 else y0b
        yv1 = y1a if ci % 2 == 0 else y1b

        @plsc.parallel_loop(0, _CHT, 1, unroll=4)
        def _acc_rows(r):
            for c in range(_D // 16):
                sl = (r, pl.ds(c * 16, 16))
                acc[sl] = acc[sl] + yv0[sl] + yv1[sl]

        pltpu.sync_copy(acc, out_hbm.at[pl.ds(t0, _CHT)])
        pending = nxt


def kernel(x, Wg, W1, W2):
    d0, d1, s0, s1, cpa, cpb = pl.pallas_call(
        _router_body,
        grid=(_T // _BT_R,),
        in_specs=[
            pl.BlockSpec((_BT_R, _D), lambda b: (b, 0)),
            pl.BlockSpec((_D, _E), lambda b: (0, 0)),
        ],
        out_specs=[
            pl.BlockSpec((_BT_R, 1), lambda b: (b, 0)),
            pl.BlockSpec((_BT_R, 1), lambda b: (b, 0)),
            pl.BlockSpec((_BT_R, 1), lambda b: (b, 0)),
            pl.BlockSpec((_BT_R, 1), lambda b: (b, 0)),
            pl.BlockSpec((_BT_R, 128), lambda b: (b, 0)),
            pl.BlockSpec((_BT_R, 128), lambda b: (b, 0)),
        ],
        out_shape=[
            jax.ShapeDtypeStruct((_T, 1), jnp.int32),
            jax.ShapeDtypeStruct((_T, 1), jnp.int32),
            jax.ShapeDtypeStruct((_T, 1), jnp.int32),
            jax.ShapeDtypeStruct((_T, 1), jnp.int32),
            jax.ShapeDtypeStruct((_T, 128), jnp.float32),
            jax.ShapeDtypeStruct((_T, 128), jnp.float32),
        ],
        scratch_shapes=[pltpu.VMEM((1, _E), jnp.float32)],
    )(x, Wg)

    mesh = plsc.VectorSubcoreMesh(core_axis_name="c", subcore_axis_name="s")

    dispatch = functools.partial(
        pl.kernel,
        mesh=mesh,
        out_type=[
            jax.ShapeDtypeStruct((_NSLOT + _BC, _D), jnp.float32),
            jax.ShapeDtypeStruct((_NSLOT + _BC, 128), jnp.float32),
        ],
        scratch_types=[
            pltpu.VMEM((_CHD,), jnp.int32),
            pltpu.VMEM((_CHD,), jnp.int32),
            pltpu.VMEM((_CHD, _D), jnp.float32),
            pltpu.VMEM((_CHD, 128), jnp.float32),
            pltpu.VMEM((_CHD, 128), jnp.float32),
            pltpu.SemaphoreType.DMA,
            pltpu.SemaphoreType.DMA,
        ],
    )(_dispatch_sc)
    d0f = d0.reshape(_T)
    d1f = d1.reshape(_T)
    s0f = s0.reshape(_T)
    s1f = s1.reshape(_T)
    buf, cslot = dispatch(x, d0f, d1f, cpa, cpb)

    y = pl.pallas_call(
        _expert_body,
        grid=(_NBLK + 1,),
        in_specs=[
            pl.BlockSpec((_BC, _D), lambda i: (i, 0)),
            pl.BlockSpec((1, _D, _H), lambda i: (jnp.minimum(i // (_CAP // _BC), _E - 1), 0, 0)),
            pl.BlockSpec((1, _H, _D), lambda i: (jnp.minimum(i // (_CAP // _BC), _E - 1), 0, 0)),
            pl.BlockSpec((_BC, 128), lambda i: (i, 0)),
        ],
        out_specs=pl.BlockSpec((_BC, _D), lambda i: (i, 0)),
        out_shape=jax.ShapeDtypeStruct((_NSLOT + _BC, _D), jnp.float32),
    )(buf, W1, W2, cslot)

    combine = functools.partial(
        pl.kernel,
        mesh=mesh,
        out_type=jax.ShapeDtypeStruct((_T, _D), jnp.float32),
        scratch_types=[
            pltpu.VMEM((_CHT,), jnp.int32),
            pltpu.VMEM((_CHT,), jnp.int32),
            pltpu.VMEM((_CHT,), jnp.int32),
            pltpu.VMEM((_CHT,), jnp.int32),
            pltpu.VMEM((_CHT, _D), jnp.float32),
            pltpu.VMEM((_CHT, _D), jnp.float32),
            pltpu.VMEM((_CHT, _D), jnp.float32),
            pltpu.VMEM((_CHT, _D), jnp.float32),
            pltpu.VMEM((_CHT, _D), jnp.float32),
            pltpu.SemaphoreType.DMA,
            pltpu.SemaphoreType.DMA,
            pltpu.SemaphoreType.DMA,
            pltpu.SemaphoreType.DMA,
        ],
    )(_combine_sc)
    out = combine(x, y, s0f, s1f)
    return out


# speedup vs baseline: 1.0209x; 1.0084x over previous
import functools

import jax
import jax.numpy as jnp
from jax.experimental import pallas as pl
from jax.experimental.pallas import tpu as pltpu
from jax.experimental.pallas import tpu_sc as plsc

_E = 8
_K = 2
_T = 4096
_D = 768
_H = _D // 2
_CAP = int(_T * _K / _E * 1.25)
_NSLOT = _E * _CAP
_NPAIR = _T * _K

_BT_R = 1024
_BC = 256
_NBLK = _NSLOT // _BC

_NW = 32
_TPW = _T // _NW
_CHD = 64
_CHT = 32


def _router_body(x_ref, wg_ref, d0_ref, d1_ref, s0_ref, s1_ref, cpa_ref, cpb_ref, carry_ref):
    b = pl.program_id(0)

    @pl.when(b == 0)
    def _():
        carry_ref[...] = jnp.zeros_like(carry_ref)

    x = x_ref[...]
    logits = jnp.dot(x, wg_ref[...], preferred_element_type=jnp.float32)

    iota_e = jax.lax.broadcasted_iota(jnp.int32, logits.shape, 1)
    m1 = jnp.max(logits, axis=1, keepdims=True)
    i1 = jnp.min(jnp.where(logits == m1, iota_e, _E), axis=1, keepdims=True)
    sel1 = iota_e == i1
    l2 = jnp.where(sel1, -jnp.inf, logits)
    m2 = jnp.max(l2, axis=1, keepdims=True)
    i2 = jnp.min(jnp.where(l2 == m2, iota_e, _E), axis=1, keepdims=True)
    sel2 = iota_e == i2

    g1 = 1.0 / (1.0 + jnp.exp(m2 - m1))
    g2 = 1.0 - g1

    cnt = sel1.astype(jnp.float32) + sel2.astype(jnp.float32)
    ii = jax.lax.broadcasted_iota(jnp.int32, (_BT_R, _BT_R), 0)
    jj = jax.lax.broadcasted_iota(jnp.int32, (_BT_R, _BT_R), 1)
    lt = (jj < ii).astype(jnp.float32)
    cum = jnp.dot(lt, cnt, preferred_element_type=jnp.float32) + carry_ref[...]
    carry_ref[...] += jnp.sum(cnt, axis=0, keepdims=True)

    pos1 = jnp.sum(jnp.where(sel1, cum, 0.0), axis=1, keepdims=True).astype(jnp.int32)
    pos2 = jnp.sum(jnp.where(sel2, cum, 0.0), axis=1, keepdims=True).astype(jnp.int32)
    keep1 = pos1 < _CAP
    keep2 = pos2 < _CAP

    slot1 = i1 * _CAP + pos1
    slot2 = i2 * _CAP + pos2
    dst1 = jnp.where(keep1, slot1, _NSLOT)
    dst2 = jnp.where(keep2, slot2, _NSLOT)
    src1 = jnp.where(keep1, slot1, _NSLOT)
    src2 = jnp.where(keep2, slot2, _NSLOT)
    c1 = jnp.where(keep1, g1, 0.0)
    c2 = jnp.where(keep2, g2, 0.0)

    d0_ref[...] = dst1
    d1_ref[...] = dst2
    s0_ref[...] = src1
    s1_ref[...] = src2
    cpa_ref[...] = jnp.broadcast_to(c1, (_BT_R, 128))
    cpb_ref[...] = jnp.broadcast_to(c2, (_BT_R, 128))


def _expert_body(buf_ref, w1_ref, w2_ref, cs_ref, y_ref):
    i = pl.program_id(0)

    @pl.when(i < _NBLK)
    def _():
        xb = buf_ref[...].astype(jnp.bfloat16)
        w1 = w1_ref[0].astype(jnp.bfloat16)
        w2 = w2_ref[0].astype(jnp.bfloat16)
        h = jnp.maximum(jnp.dot(xb, w1, preferred_element_type=jnp.float32), 0.0)
        h2 = (h * cs_ref[:, 0:1]).astype(jnp.bfloat16)
        y_ref[...] = jnp.maximum(
            jnp.dot(h2, w2, preferred_element_type=jnp.float32), 0.0)

    @pl.when(i == _NBLK)
    def _():
        y_ref[...] = jnp.zeros_like(y_ref)


def _dispatch_sc(x_hbm, d0_hbm, d1_hbm, cpa_hbm, cpb_hbm, buf_hbm, cslot_hbm,
                 d0v, d1v, rows, cv0, cv1, sem1, sem2):
    wid = jax.lax.axis_index("s") * 2 + jax.lax.axis_index("c")
    for ci in range(_TPW // _CHD):
        t0 = wid * _TPW + ci * _CHD
        pltpu.sync_copy(d0_hbm.at[pl.ds(t0, _CHD)], d0v)
        pltpu.sync_copy(d1_hbm.at[pl.ds(t0, _CHD)], d1v)
        cpx = pltpu.async_copy(x_hbm.at[pl.ds(t0, _CHD)], rows, sem1)
        cpa = pltpu.async_copy(cpa_hbm.at[pl.ds(t0, _CHD)], cv0, sem1)
        cpb = pltpu.async_copy(cpb_hbm.at[pl.ds(t0, _CHD)], cv1, sem1)
        cpx.wait(); cpa.wait(); cpb.wait()
        s0 = pltpu.async_copy(rows, buf_hbm.at[d0v], sem2)
        s1 = pltpu.async_copy(rows, buf_hbm.at[d1v], sem2)
        s2 = pltpu.async_copy(cv0, cslot_hbm.at[d0v], sem2)
        s3 = pltpu.async_copy(cv1, cslot_hbm.at[d1v], sem2)
        s0.wait(); s1.wait(); s2.wait(); s3.wait()


def _combine_sc(x_hbm, y_hbm, s0_hbm, s1_hbm, out_hbm,
                s0a, s1a, s0b, s1b, acc, y0a, y1a, y0b, y1b,
                semA0, semA1, semB0, semB1):
    wid = jax.lax.axis_index("s") * 2 + jax.lax.axis_index("c")
    nch = _TPW // _CHT

    def _load_src(ci):
        if ci % 2 == 0:
            s0v, s1v, b0, b1, m0, m1 = s0a, s1a, y0a, y1a, semA0, semA1
        else:
            s0v, s1v, b0, b1, m0, m1 = s0b, s1b, y0b, y1b, semB0, semB1
        t0 = wid * _TPW + ci * _CHT
        pltpu.sync_copy(s0_hbm.at[pl.ds(t0, _CHT)], s0v)
        pltpu.sync_copy(s1_hbm.at[pl.ds(t0, _CHT)], s1v)
        ga = pltpu.async_copy(y_hbm.at[s0v], b0, m0)
        gb = pltpu.async_copy(y_hbm.at[s1v], b1, m1)
        return ga, gb

    pending = _load_src(0)
    for ci in range(nch):
        nxt = _load_src(ci + 1) if ci + 1 < nch else None
        t0 = wid * _TPW + ci * _CHT
        pltpu.sync_copy(x_hbm.at[pl.ds(t0, _CHT)], acc)
        ga, gb = pending
        ga.wait()
        gb.wait()
        yv0 = y0a if ci % 2 == 0 else y0b
        yv1 = y1a if ci % 2 == 0 else y1b

        @plsc.parallel_loop(0, _CHT, 1, unroll=4)
        def _acc_rows(r):
            for c in range(_D // 16):
                sl = (r, pl.ds(c * 16, 16))
                acc[sl] = acc[sl] + yv0[sl] + yv1[sl]

        pltpu.sync_copy(acc, out_hbm.at[pl.ds(t0, _CHT)])
        pending = nxt


def kernel(x, Wg, W1, W2):
    d0, d1, s0, s1, cpa, cpb = pl.pallas_call(
        _router_body,
        grid=(_T // _BT_R,),
        in_specs=[
            pl.BlockSpec((_BT_R, _D), lambda b: (b, 0)),
            pl.BlockSpec((_D, _E), lambda b: (0, 0)),
        ],
        out_specs=[
            pl.BlockSpec((_BT_R, 1), lambda b: (b, 0)),
            pl.BlockSpec((_BT_R, 1), lambda b: (b, 0)),
            pl.BlockSpec((_BT_R, 1), lambda b: (b, 0)),
            pl.BlockSpec((_BT_R, 1), lambda b: (b, 0)),
            pl.BlockSpec((_BT_R, 128), lambda b: (b, 0)),
            pl.BlockSpec((_BT_R, 128), lambda b: (b, 0)),
        ],
        out_shape=[
            jax.ShapeDtypeStruct((_T, 1), jnp.int32),
            jax.ShapeDtypeStruct((_T, 1), jnp.int32),
            jax.ShapeDtypeStruct((_T, 1), jnp.int32),
            jax.ShapeDtypeStruct((_T, 1), jnp.int32),
            jax.ShapeDtypeStruct((_T, 128), jnp.float32),
            jax.ShapeDtypeStruct((_T, 128), jnp.float32),
        ],
        scratch_shapes=[pltpu.VMEM((1, _E), jnp.float32)],
    )(x, Wg)

    mesh = plsc.VectorSubcoreMesh(core_axis_name="c", subcore_axis_name="s")

    dispatch = functools.partial(
        pl.kernel,
        mesh=mesh,
        out_type=[
            jax.ShapeDtypeStruct((_NSLOT + _BC, _D), jnp.float32),
            jax.ShapeDtypeStruct((_NSLOT + _BC, 128), jnp.float32),
        ],
        scratch_types=[
            pltpu.VMEM((_CHD,), jnp.int32),
            pltpu.VMEM((_CHD,), jnp.int32),
            pltpu.VMEM((_CHD, _D), jnp.float32),
            pltpu.VMEM((_CHD, 128), jnp.float32),
            pltpu.VMEM((_CHD, 128), jnp.float32),
            pltpu.SemaphoreType.DMA,
            pltpu.SemaphoreType.DMA,
        ],
    )(_dispatch_sc)
    d0f = d0.reshape(_T)
    d1f = d1.reshape(_T)
    s0f = s0.reshape(_T)
    s1f = s1.reshape(_T)
    buf, cslot = dispatch(x, d0f, d1f, cpa, cpb)

    y = pl.pallas_call(
        _expert_body,
        grid=(_NBLK + 1,),
        in_specs=[
            pl.BlockSpec((_BC, _D), lambda i: (i, 0)),
            pl.BlockSpec((1, _D, _H), lambda i: (jnp.minimum(i // (_CAP // _BC), _E - 1), 0, 0)),
            pl.BlockSpec((1, _H, _D), lambda i: (jnp.minimum(i // (_CAP // _BC), _E - 1), 0, 0)),
            pl.BlockSpec((_BC, 128), lambda i: (i, 0)),
        ],
        out_specs=pl.BlockSpec((_BC, _D), lambda i: (i, 0)),
        out_shape=jax.ShapeDtypeStruct((_NSLOT + _BC, _D), jnp.float32),
    )(buf, W1, W2, cslot)

    combine = functools.partial(
        pl.kernel,
        mesh=mesh,
        out_type=jax.ShapeDtypeStruct((_T, _D), jnp.float32),
        scratch_types=[
            pltpu.VMEM((_CHT,), jnp.int32),
            pltpu.VMEM((_CHT,), jnp.int32),
            pltpu.VMEM((_CHT,), jnp.int32),
            pltpu.VMEM((_CHT,), jnp.int32),
            pltpu.VMEM((_CHT, _D), jnp.float32),
            pltpu.VMEM((_CHT, _D), jnp.float32),
            pltpu.VMEM((_CHT, _D), jnp.float32),
            pltpu.VMEM((_CHT, _D), jnp.float32),
            pltpu.VMEM((_CHT, _D), jnp.float32),
            pltpu.SemaphoreType.DMA,
            pltpu.SemaphoreType.DMA,
            pltpu.SemaphoreType.DMA,
            pltpu.SemaphoreType.DMA,
        ],
    )(_combine_sc)
    out = combine(x, y, s0f, s1f)
    return out
